# Initial kernel scaffold; baseline (speedup 1.0000x reference)
import jax
import jax.numpy as jnp
from jax.experimental import pallas as pl


def kernel(x):
    raise NotImplementedError("write your pallas kernel here")


# speedup vs baseline: 5.2443x; 5.2443x over previous
import jax
import jax.numpy as jnp
from jax.experimental import pallas as pl
from jax.experimental.pallas import tpu as pltpu

_R = 512
_C = 1024
_SUB = 128


def _scan_kernel(x_ref, o_ref, carry_ref):
    ci = pl.program_id(1)

    @pl.when(ci == 0)
    def _():
        carry_ref[...] = jnp.zeros_like(carry_ref)

    x = x_ref[...]
    T = (jax.lax.broadcasted_iota(jnp.int32, (_SUB, _SUB), 0)
         < jax.lax.broadcasted_iota(jnp.int32, (_SUB, _SUB), 1)
         ).astype(jnp.float32)
    carry = carry_ref[...]
    for k in range(_C // _SUB):
        xs = x[:, k * _SUB:(k + 1) * _SUB]
        excl = jnp.dot(xs, T, preferred_element_type=jnp.float32)
        o_ref[:, k * _SUB:(k + 1) * _SUB] = excl + carry
        carry = carry + jnp.sum(xs, axis=1, keepdims=True)
    carry_ref[...] = carry


@jax.jit
def kernel(x):
    m, n = x.shape
    grid = (m // _R, n // _C)
    return pl.pallas_call(
        _scan_kernel,
        grid=grid,
        in_specs=[pl.BlockSpec((_R, _C), lambda i, j: (i, j))],
        out_specs=pl.BlockSpec((_R, _C), lambda i, j: (i, j)),
        out_shape=jax.ShapeDtypeStruct((m, n), x.dtype),
        scratch_shapes=[pltpu.VMEM((_R, 1), jnp.float32)],
        compiler_params=pltpu.CompilerParams(
            dimension_semantics=("parallel", "arbitrary")),
    )(x)


# speedup vs baseline: 6.1314x; 1.1692x over previous
import jax
import jax.numpy as jnp
from jax.experimental import pallas as pl
from jax.experimental.pallas import tpu as pltpu

_R = 512
_C = 1024
_SUB = 128


def _scan_kernel(x_ref, o_ref, carry_ref):
    ci = pl.program_id(1)

    @pl.when(ci == 0)
    def _():
        carry_ref[...] = jnp.zeros_like(carry_ref)

    x = x_ref[...]
    T = (jax.lax.broadcasted_iota(jnp.int32, (_SUB, _SUB), 0)
         < jax.lax.broadcasted_iota(jnp.int32, (_SUB, _SUB), 1)
         ).astype(jnp.float32)
    ones = jnp.ones((_SUB, _SUB), jnp.float32)
    carry = carry_ref[...]
    for k in range(_C // _SUB):
        xs = x[:, k * _SUB:(k + 1) * _SUB]
        excl = jnp.dot(xs, T, preferred_element_type=jnp.float32)
        o_ref[:, k * _SUB:(k + 1) * _SUB] = excl + carry
        carry = carry + jnp.dot(xs, ones, preferred_element_type=jnp.float32)
    carry_ref[...] = carry


@jax.jit
def kernel(x):
    m, n = x.shape
    grid = (m // _R, n // _C)
    return pl.pallas_call(
        _scan_kernel,
        grid=grid,
        in_specs=[pl.BlockSpec((_R, _C), lambda i, j: (i, j))],
        out_specs=pl.BlockSpec((_R, _C), lambda i, j: (i, j)),
        out_shape=jax.ShapeDtypeStruct((m, n), x.dtype),
        scratch_shapes=[pltpu.VMEM((_R, _SUB), jnp.float32)],
        compiler_params=pltpu.CompilerParams(
            dimension_semantics=("parallel", "arbitrary")),
    )(x)
